# baseline (speedup 1.0000x reference)
import functools

import jax
import jax.numpy as jnp
from jax import lax
from jax.experimental import pallas as pl
from jax.experimental.pallas import tpu as pltpu
from jax.experimental.pallas import tpu_sc as plsc

D = 64
NC = 2
NS = 16
NW = NC * NS
BLK = 128


@functools.partial(jax.jit, static_argnames=("n_tok", "n_pos"))
def _emb_lookup(xf, table, n_tok, n_pos):
    n_iblk = n_tok // BLK
    iblk_per_w = n_iblk // NW
    span_len = BLK * n_pos
    mesh = plsc.VectorSubcoreMesh(core_axis_name="c", subcore_axis_name="s")

    @functools.partial(
        pl.kernel,
        mesh=mesh,
        out_type=jax.ShapeDtypeStruct(
            (n_pos, D // 8, n_iblk, 8, BLK), jnp.float32),
        scratch_types=[
            pltpu.VMEM((span_len,), jnp.int32),
            pltpu.VMEM((2, BLK), jnp.int32),
            pltpu.VMEM((2, BLK, D), jnp.float32),
            pltpu.VMEM((2, D, BLK), jnp.float32),
            pltpu.SemaphoreType.DMA,
            pltpu.SemaphoreType.DMA,
            pltpu.SemaphoreType.DMA,
            pltpu.SemaphoreType.DMA,
        ],
        compiler_params=pltpu.CompilerParams(
            use_tc_tiling_on_sc=False, needs_layout_passes=False),
    )
    def emb(x_hbm, table_hbm, out_hbm, span_v, idx_v, rows_v, trans_v,
            gsem0, gsem1, osem0, osem1):
        wid = lax.axis_index("s") * NC + lax.axis_index("c")
        gsems = (gsem0, gsem1)
        osems = (osem0, osem1)
        iota16 = jnp.arange(16, dtype=jnp.int32)
        iota_pos = iota16 * n_pos

        def stage_a(j, b):
            for t in range(BLK // 16):
                pos = iota_pos + (t * 16 * n_pos + j)
                v = plsc.load_gather(span_v, [pos])
                idx_v[b, pl.ds(t * 16, 16)] = v
            pltpu.async_copy(table_hbm.at[idx_v.at[b]], rows_v.at[b],
                             gsems[b])

        def drain_out(b):
            for kh in range(D // 8):
                pltpu.make_async_copy(
                    trans_v.at[b, pl.ds(kh * 8, 8), :],
                    out_hbm.at[0, kh, 0], osems[b]).wait()

        def stage_b(j, ih, b, first):
            pltpu.make_async_copy(table_hbm.at[idx_v.at[b]], rows_v.at[b],
                                  gsems[b]).wait()
            if first:
                @pl.when(j >= 2)
                def _():
                    drain_out(b)
            else:
                drain_out(b)

            @plsc.parallel_loop(0, BLK, unroll=4)
            def tr(il):
                col = jnp.full((16,), il, dtype=jnp.int32)
                for t in range(D // 16):
                    row = iota16 + (t * 16)
                    v = rows_v[b, il, pl.ds(t * 16, 16)]
                    plsc.store_scatter(trans_v.at[b], [row, col], v)
            for kh in range(D // 8):
                pltpu.async_copy(trans_v.at[b, pl.ds(kh * 8, 8), :],
                                 out_hbm.at[j, kh, ih], osems[b])

        for ihb in range(iblk_per_w):
            ih = wid * iblk_per_w + ihb
            pltpu.sync_copy(x_hbm.at[pl.ds(ih * span_len, span_len)], span_v)
            stage_a(0, 0)

            def pair(p, carry):
                j0 = 2 * p
                stage_a(j0 + 1, 1)
                stage_b(j0, ih, 0, first=True)

                @pl.when(p < n_pos // 2 - 1)
                def _():
                    stage_a(j0 + 2, 0)

                stage_b(j0 + 1, ih, 1, first=True)
                return carry

            lax.fori_loop(0, n_pos // 2, pair, 0)
            drain_out(0)
            drain_out(1)

    return emb(xf, table)


def kernel(x, table):
    n_tok, n_pos = x.shape
    xf = x.reshape(-1).astype(jnp.int32)
    out5 = _emb_lookup(xf, table, n_tok, n_pos)
    return out5.transpose(2, 4, 0, 1, 3).reshape(n_tok, n_pos, D)


# speedup vs baseline: 1.6599x; 1.6599x over previous
import functools

import jax
import jax.numpy as jnp
from jax import lax
from jax.experimental import pallas as pl
from jax.experimental.pallas import tpu as pltpu
from jax.experimental.pallas import tpu_sc as plsc

D = 64
NC = 2
NS = 16
NW = NC * NS
BLK = 128


@functools.partial(jax.jit, static_argnames=("n_tok", "n_pos"))
def _emb_lookup(xf, table, n_tok, n_pos):
    n_iblk = n_tok // BLK
    iblk_per_w = n_iblk // NW
    span_len = BLK * n_pos
    mesh = plsc.VectorSubcoreMesh(core_axis_name="c", subcore_axis_name="s")

    @functools.partial(
        pl.kernel,
        mesh=mesh,
        out_type=jax.ShapeDtypeStruct(
            (n_pos, D // 8, n_iblk, 8, BLK), jnp.float32),
        scratch_types=[
            pltpu.VMEM((span_len,), jnp.int32),
            pltpu.VMEM((2, BLK), jnp.int32),
            pltpu.VMEM((2, BLK, D), jnp.float32),
            pltpu.VMEM((2, D, BLK + 1), jnp.float32),
            pltpu.SemaphoreType.DMA,
            pltpu.SemaphoreType.DMA,
            pltpu.SemaphoreType.DMA,
            pltpu.SemaphoreType.DMA,
        ],
        compiler_params=pltpu.CompilerParams(
            use_tc_tiling_on_sc=False, needs_layout_passes=False),
    )
    def emb(x_hbm, table_hbm, out_hbm, span_v, idx_v, rows_v, trans_v,
            gsem0, gsem1, osem0, osem1):
        wid = lax.axis_index("s") * NC + lax.axis_index("c")
        gsems = (gsem0, gsem1)
        osems = (osem0, osem1)
        iota16 = jnp.arange(16, dtype=jnp.int32)
        iota_pos = iota16 * n_pos

        def stage_a(j, b):
            for t in range(BLK // 16):
                pos = iota_pos + (t * 16 * n_pos + j)
                v = plsc.load_gather(span_v, [pos])
                idx_v[b, pl.ds(t * 16, 16)] = v
            pltpu.async_copy(table_hbm.at[idx_v.at[b]], rows_v.at[b],
                             gsems[b])

        def drain_out(b):
            for kh in range(D // 8):
                pltpu.make_async_copy(
                    trans_v.at[b, pl.ds(kh * 8, 8), pl.ds(0, BLK)],
                    out_hbm.at[0, kh, 0], osems[b]).wait()

        def stage_b(j, ih, b, first):
            pltpu.make_async_copy(table_hbm.at[idx_v.at[b]], rows_v.at[b],
                                  gsems[b]).wait()
            if first:
                @pl.when(j >= 2)
                def _():
                    drain_out(b)
            else:
                drain_out(b)

            @plsc.parallel_loop(0, BLK, unroll=4)
            def tr(il):
                col = jnp.full((16,), il, dtype=jnp.int32)
                for t in range(D // 16):
                    row = iota16 + (t * 16)
                    v = rows_v[b, il, pl.ds(t * 16, 16)]
                    plsc.store_scatter(trans_v.at[b], [row, col], v)
            for kh in range(D // 8):
                pltpu.async_copy(trans_v.at[b, pl.ds(kh * 8, 8), pl.ds(0, BLK)],
                                 out_hbm.at[j, kh, ih], osems[b])

        for ihb in range(iblk_per_w):
            ih = wid * iblk_per_w + ihb
            pltpu.sync_copy(x_hbm.at[pl.ds(ih * span_len, span_len)], span_v)
            stage_a(0, 0)

            def pair(p, carry):
                j0 = 2 * p
                stage_a(j0 + 1, 1)
                stage_b(j0, ih, 0, first=True)

                @pl.when(p < n_pos // 2 - 1)
                def _():
                    stage_a(j0 + 2, 0)

                stage_b(j0 + 1, ih, 1, first=True)
                return carry

            lax.fori_loop(0, n_pos // 2, pair, 0)
            drain_out(0)
            drain_out(1)

    return emb(xf, table)


def kernel(x, table):
    n_tok, n_pos = x.shape
    xf = x.reshape(-1).astype(jnp.int32)
    out5 = _emb_lookup(xf, table, n_tok, n_pos)
    return out5.transpose(2, 4, 0, 1, 3).reshape(n_tok, n_pos, D)


# speedup vs baseline: 2.1980x; 1.3242x over previous
import functools

import jax
import jax.numpy as jnp
from jax import lax
from jax.experimental import pallas as pl
from jax.experimental.pallas import tpu as pltpu
from jax.experimental.pallas import tpu_sc as plsc

D = 64
NC = 2
NS = 16
NW = NC * NS
BLK = 128
RB = 8192


def _relayout(tableT, n_rows):
    nblk = pl.cdiv(n_rows, RB)

    def body(t_ref, o_ref):
        tT = t_ref[...].T
        v = tT.reshape(RB // 2, 2, D)
        o_ref[:, 0:D] = v[:, 0, :]
        o_ref[:, D:2 * D] = v[:, 1, :]

    return pl.pallas_call(
        body,
        grid=(nblk,),
        in_specs=[pl.BlockSpec((D, RB), lambda r: (0, r))],
        out_specs=pl.BlockSpec((RB // 2, 2 * D), lambda r: (r, 0)),
        out_shape=jax.ShapeDtypeStruct((nblk * RB // 2, 2 * D), jnp.float32),
    )(tableT)


@functools.partial(jax.jit, static_argnames=("n_tok", "n_pos"))
def _emb_lookup(xf, table, n_tok, n_pos):
    n_iblk = n_tok // BLK
    iblk_per_w = n_iblk // NW
    span_len = BLK * n_pos
    n_rows = table.shape[0]
    tab_lin = _relayout(table.T, n_rows).reshape(-1, D)
    mesh = plsc.VectorSubcoreMesh(core_axis_name="c", subcore_axis_name="s")

    @functools.partial(
        pl.kernel,
        mesh=mesh,
        out_type=jax.ShapeDtypeStruct(
            (n_pos, D // 8, n_iblk, 8, BLK), jnp.float32),
        scratch_types=[
            pltpu.VMEM((span_len,), jnp.int32),
            pltpu.VMEM((2, BLK), jnp.int32),
            pltpu.VMEM((2, BLK, D), jnp.float32),
            pltpu.VMEM((2, D, BLK + 1), jnp.float32),
            pltpu.SemaphoreType.DMA,
            pltpu.SemaphoreType.DMA,
            pltpu.SemaphoreType.DMA,
            pltpu.SemaphoreType.DMA,
        ],
        compiler_params=pltpu.CompilerParams(
            use_tc_tiling_on_sc=False, needs_layout_passes=False),
    )
    def emb(x_hbm, table_hbm, out_hbm, span_v, idx_v, rows_v, trans_v,
            gsem0, gsem1, osem0, osem1):
        wid = lax.axis_index("s") * NC + lax.axis_index("c")
        gsems = (gsem0, gsem1)
        osems = (osem0, osem1)
        iota16 = jnp.arange(16, dtype=jnp.int32)
        iota_pos = iota16 * n_pos

        def stage_a(j, b):
            for t in range(BLK // 16):
                pos = iota_pos + (t * 16 * n_pos + j)
                v = plsc.load_gather(span_v, [pos])
                idx_v[b, pl.ds(t * 16, 16)] = v
            pltpu.async_copy(table_hbm.at[idx_v.at[b]], rows_v.at[b],
                             gsems[b])

        def drain_out(b):
            for kh in range(D // 8):
                pltpu.make_async_copy(
                    trans_v.at[b, pl.ds(kh * 8, 8), pl.ds(0, BLK)],
                    out_hbm.at[0, kh, 0], osems[b]).wait()

        def stage_b(j, ih, b, first):
            pltpu.make_async_copy(table_hbm.at[idx_v.at[b]], rows_v.at[b],
                                  gsems[b]).wait()
            if first:
                @pl.when(j >= 2)
                def _():
                    drain_out(b)
            else:
                drain_out(b)

            @plsc.parallel_loop(0, BLK, unroll=4)
            def tr(il):
                col = jnp.full((16,), il, dtype=jnp.int32)
                for t in range(D // 16):
                    row = iota16 + (t * 16)
                    v = rows_v[b, il, pl.ds(t * 16, 16)]
                    plsc.store_scatter(trans_v.at[b], [row, col], v)
            for kh in range(D // 8):
                pltpu.async_copy(trans_v.at[b, pl.ds(kh * 8, 8), pl.ds(0, BLK)],
                                 out_hbm.at[j, kh, ih], osems[b])

        for ihb in range(iblk_per_w):
            ih = wid * iblk_per_w + ihb
            pltpu.sync_copy(x_hbm.at[pl.ds(ih * span_len, span_len)], span_v)
            stage_a(0, 0)

            def pair(p, carry):
                j0 = 2 * p
                stage_a(j0 + 1, 1)
                stage_b(j0, ih, 0, first=True)

                @pl.when(p < n_pos // 2 - 1)
                def _():
                    stage_a(j0 + 2, 0)

                stage_b(j0 + 1, ih, 1, first=True)
                return carry

            lax.fori_loop(0, n_pos // 2, pair, 0)
            drain_out(0)
            drain_out(1)

    return emb(xf, tab_lin)


def kernel(x, table):
    n_tok, n_pos = x.shape
    xf = x.reshape(-1).astype(jnp.int32)
    out5 = _emb_lookup(xf, table, n_tok, n_pos)
    return out5.transpose(2, 4, 0, 1, 3).reshape(n_tok, n_pos, D)
